# baseline (speedup 1.0000x reference)
import jax
import jax.numpy as jnp
from jax.experimental import pallas as pl
from jax.experimental.pallas import tpu as pltpu

NUM_TOKENS = 32768
HIDDEN = 768
NUM_EXPERTS = 64

BLOCK = 2048
DEPTH = 4
SUB = 4
ROWS = BLOCK // SUB


def _router_block(x_hbm, wt_ref, scores_ref, w_ref, i_ref, xbuf, sems):
    step = pl.program_id(0)
    nsteps = pl.num_programs(0)

    def copy(block, slot, j):
        return pltpu.make_async_copy(
            x_hbm.at[pl.ds(block * BLOCK + j * ROWS, ROWS), :],
            xbuf.at[slot, pl.ds(j * ROWS, ROWS), :],
            sems.at[slot],
        )

    def start_all(block, slot):
        for j in range(SUB):
            copy(block, slot, j).start()

    @pl.when(step == 0)
    def _():
        for d in range(DEPTH):
            start_all(d, d)

    slot = jax.lax.rem(step, DEPTH)
    for j in range(SUB):
        copy(step, slot, j).wait()

    logits = jnp.dot(xbuf[slot], wt_ref[...], preferred_element_type=jnp.float32)
    m = jnp.max(logits, axis=-1, keepdims=True)
    e = jnp.exp(logits - m)
    s = jnp.sum(e, axis=-1, keepdims=True)
    scores_ref[...] = e / s
    w_ref[...] = 1.0 / s
    lane = jax.lax.broadcasted_iota(jnp.int32, logits.shape, 1).astype(jnp.float32)
    hit = jnp.where(logits == m, lane, float(NUM_EXPERTS))
    i_ref[...] = jnp.min(hit, axis=-1, keepdims=True).astype(jnp.int32)

    @pl.when(step + DEPTH < nsteps)
    def _():
        start_all(step + DEPTH, slot)


@jax.jit
def _router(x, Wt):
    n_blocks = NUM_TOKENS // BLOCK
    scores, w, idx = pl.pallas_call(
        _router_block,
        grid=(n_blocks,),
        in_specs=[
            pl.BlockSpec(memory_space=pl.MemorySpace.ANY),
            pl.BlockSpec((HIDDEN, NUM_EXPERTS), lambda i: (0, 0)),
        ],
        out_specs=[
            pl.BlockSpec((BLOCK, NUM_EXPERTS), lambda i: (i, 0)),
            pl.BlockSpec((BLOCK, 1), lambda i: (i, 0)),
            pl.BlockSpec((BLOCK, 1), lambda i: (i, 0)),
        ],
        out_shape=[
            jax.ShapeDtypeStruct((NUM_TOKENS, NUM_EXPERTS), jnp.float32),
            jax.ShapeDtypeStruct((NUM_TOKENS, 1), jnp.float32),
            jax.ShapeDtypeStruct((NUM_TOKENS, 1), jnp.int32),
        ],
        scratch_shapes=[
            pltpu.VMEM((DEPTH, BLOCK, HIDDEN), jnp.float32),
            pltpu.SemaphoreType.DMA((DEPTH,)),
        ],
        compiler_params=pltpu.CompilerParams(
            dimension_semantics=("arbitrary",),
        ),
    )(x, Wt)
    return w, idx, scores


def kernel(x, W):
    x2 = x.reshape(-1, x.shape[-1])
    w, idx, scores = _router(x2, W.T)
    return (w, idx, scores)
